# Initial kernel scaffold; baseline (speedup 1.0000x reference)
import jax
import jax.numpy as jnp
from jax.experimental import pallas as pl


def kernel(x, edge_index, edge_attr, W_e1, b_e1, W_e2, b_e2, g_e, be_ln, W_n1, b_n1, W_n2, b_n2, g_n, bn_ln):
    raise NotImplementedError("write your pallas kernel here")


# speedup vs baseline: 1.8882x; 1.8882x over previous
import functools

import jax
import jax.numpy as jnp
from jax import lax
from jax.experimental import pallas as pl
from jax.experimental.pallas import tpu as pltpu
from jax.experimental.pallas import tpu_sc as plsc

_NC = 2
_NS = 16
_NW = _NC * _NS

_EP = 327680
_IW = 40
_CH = 8


def _silu(z):
    return z * jax.nn.sigmoid(z)


def _layer_norm(z, g, b):
    m = jnp.mean(z, axis=-1, keepdims=True)
    v = jnp.mean((z - m) ** 2, axis=-1, keepdims=True)
    return (z - m) * lax.rsqrt(v + 1e-5) * g + b


def _gather_call(x, src2, dst2):
    N, D = x.shape
    R, IW = src2.shape
    E = R * IW
    RW = R // _NW
    NIT = RW // _CH
    CE = _CH * IW

    mesh = plsc.VectorSubcoreMesh(core_axis_name="c", subcore_axis_name="s")

    @functools.partial(
        pl.kernel,
        out_type=[jax.ShapeDtypeStruct((E, D), jnp.float32),
                  jax.ShapeDtypeStruct((E, D), jnp.float32)],
        mesh=mesh,
        scratch_types=[
            pltpu.VMEM((_CH, IW), jnp.int32),
            pltpu.VMEM((_CH, IW), jnp.int32),
            pltpu.VMEM((CE, D), jnp.float32),
            pltpu.VMEM((CE, D), jnp.float32),
            pltpu.SemaphoreType.DMA,
        ],
    )
    def gather_kernel(x_hbm, src_hbm, dst_hbm, xs_hbm, xd_hbm,
                      idx_s, idx_d, rows_s, rows_d, sem):
        wid = lax.axis_index("s") * _NC + lax.axis_index("c")

        @pl.loop(0, NIT)
        def _chunk(i):
            row0 = wid * RW + i * _CH
            pltpu.sync_copy(src_hbm.at[pl.ds(row0, _CH)], idx_s)
            pltpu.sync_copy(dst_hbm.at[pl.ds(row0, _CH)], idx_d)
            copies = []
            for j in range(_CH):
                copies.append(pltpu.async_copy(
                    x_hbm.at[idx_s.at[j]], rows_s.at[pl.ds(j * IW, IW)], sem))
                copies.append(pltpu.async_copy(
                    x_hbm.at[idx_d.at[j]], rows_d.at[pl.ds(j * IW, IW)], sem))
            for c in copies:
                c.wait()
            e0 = row0 * IW
            pltpu.sync_copy(rows_s, xs_hbm.at[pl.ds(e0, CE)])
            pltpu.sync_copy(rows_d, xd_hbm.at[pl.ds(e0, CE)])

    return gather_kernel(x, src2, dst2)


def _scatter_call(h, dst2, zeros, N):
    E, Hd = h.shape
    R, IW = dst2.shape
    HC = Hd // _NC
    ZR = zeros.shape[0]
    PADN = ZR * _NS
    RW = R // _NS
    NIT = RW // _CH
    CE = _CH * IW

    mesh = plsc.VectorSubcoreMesh(core_axis_name="c", subcore_axis_name="s")

    @functools.partial(
        pl.kernel,
        out_type=jax.ShapeDtypeStruct((_NC * PADN, HC), jnp.float32),
        mesh=mesh,
        scratch_types=[
            pltpu.VMEM_SHARED((PADN, HC), jnp.float32),
            pltpu.VMEM((_CH, IW), jnp.int32),
            pltpu.VMEM((CE, HC), jnp.float32),
        ],
    )
    def scatter_kernel(h_hbm, dst_hbm, zero_hbm, agg_hbm,
                       acc, idx_v, rows_v):
        c = lax.axis_index("c")
        s = lax.axis_index("s")

        pltpu.sync_copy(zero_hbm, acc.at[pl.ds(s * ZR, ZR)])
        plsc.subcore_barrier()

        @pl.loop(0, NIT)
        def _chunk(i):
            row0 = s * RW + i * _CH
            pltpu.sync_copy(dst_hbm.at[pl.ds(row0, _CH)], idx_v)
            pltpu.sync_copy(h_hbm.at[pl.ds(row0 * IW, CE), pl.ds(c * HC, HC)],
                            rows_v)
            for j in range(_CH):
                pltpu.sync_copy(rows_v.at[pl.ds(j * IW, IW)],
                                acc.at[idx_v.at[j]], add=True)

        plsc.subcore_barrier()
        pltpu.sync_copy(acc.at[pl.ds(s * ZR, ZR)],
                        agg_hbm.at[pl.ds(c * PADN + s * ZR, ZR)])

    return scatter_kernel(h, dst2, zeros)


def _edge_mlp_call(xs, xd, ea, W1s, W1d, W1e, b1, W2, b2, g, be):
    E, D = xs.shape
    Hd = W2.shape[1]
    BE = 2048
    grid = (E // BE,)

    def body(xs_ref, xd_ref, ea_ref, W1s_ref, W1d_ref, W1e_ref, b1_ref,
             W2_ref, b2_ref, g_ref, be_ref, h_ref):
        f32 = jnp.float32
        z = (jnp.dot(xs_ref[...].astype(jnp.bfloat16), W1s_ref[...],
                     preferred_element_type=f32)
             + jnp.dot(xd_ref[...].astype(jnp.bfloat16), W1d_ref[...],
                       preferred_element_type=f32)
             + jnp.dot(ea_ref[...].astype(jnp.bfloat16), W1e_ref[...],
                       preferred_element_type=f32)
             + b1_ref[...])
        h1 = _silu(z).astype(jnp.bfloat16)
        h2 = jnp.dot(h1, W2_ref[...], preferred_element_type=f32) + b2_ref[...]
        h_ref[...] = _layer_norm(h2, g_ref[...], be_ref[...])

    full = lambda a: pl.BlockSpec(a.shape, lambda i: (0,) * a.ndim)
    return pl.pallas_call(
        body,
        grid=grid,
        in_specs=[
            pl.BlockSpec((BE, D), lambda i: (i, 0)),
            pl.BlockSpec((BE, D), lambda i: (i, 0)),
            pl.BlockSpec((BE, ea.shape[1]), lambda i: (i, 0)),
            full(W1s), full(W1d), full(W1e), full(b1),
            full(W2), full(b2), full(g), full(be),
        ],
        out_specs=pl.BlockSpec((BE, Hd), lambda i: (i, 0)),
        out_shape=jax.ShapeDtypeStruct((E, Hd), jnp.float32),
    )(xs, xd, ea, W1s, W1d, W1e, b1, W2, b2, g, be)


def _node_mlp_call(x, agg, Wnx, Wna, bn1, Wn2, bn2, g, b):
    N, D = x.shape
    Hd = agg.shape[1]
    BN = 1000
    grid = (N // BN,)

    def body(x_ref, agg_ref, Wnx_ref, Wna_ref, bn1_ref, Wn2_ref, bn2_ref,
             g_ref, b_ref, o_ref):
        f32 = jnp.float32
        z = (jnp.dot(x_ref[...].astype(jnp.bfloat16), Wnx_ref[...],
                     preferred_element_type=f32)
             + jnp.dot(agg_ref[...].astype(jnp.bfloat16), Wna_ref[...],
                       preferred_element_type=f32)
             + bn1_ref[...])
        u1 = _silu(z).astype(jnp.bfloat16)
        u = jnp.dot(u1, Wn2_ref[...], preferred_element_type=f32) + bn2_ref[...]
        o_ref[...] = x_ref[...] + _layer_norm(u, g_ref[...], b_ref[...])

    full = lambda a: pl.BlockSpec(a.shape, lambda i: (0,) * a.ndim)
    return pl.pallas_call(
        body,
        grid=grid,
        in_specs=[
            pl.BlockSpec((BN, D), lambda i: (i, 0)),
            pl.BlockSpec((BN, Hd), lambda i: (i, 0)),
            full(Wnx), full(Wna), full(bn1), full(Wn2), full(bn2),
            full(g), full(b),
        ],
        out_specs=pl.BlockSpec((BN, D), lambda i: (i, 0)),
        out_shape=jax.ShapeDtypeStruct((N, D), jnp.float32),
    )(x, agg, Wnx, Wna, bn1, Wn2, bn2, g, b)


def kernel(x, edge_index, edge_attr, W_e1, b_e1, W_e2, b_e2, g_e, be_ln,
           W_n1, b_n1, W_n2, b_n2, g_n, bn_ln):
    N, D = x.shape
    E = edge_index.shape[1]
    DE = edge_attr.shape[1]
    Hd = W_e2.shape[0]
    bf16 = jnp.bfloat16

    PAD = _EP - E
    R = _EP // _IW
    src = edge_index[0]
    dst = edge_index[1]
    pad0 = jnp.zeros((PAD,), jnp.int32)
    src2 = jnp.concatenate([src, pad0]).reshape(R, _IW)
    dstg2 = jnp.concatenate([dst, pad0]).reshape(R, _IW)
    dsts2 = jnp.concatenate([dst, jnp.full((PAD,), N, jnp.int32)]).reshape(R, _IW)
    ea_p = jnp.concatenate([edge_attr, jnp.zeros((PAD, DE), edge_attr.dtype)])

    xs, xd = _gather_call(x, src2, dstg2)

    W1s = W_e1[:D].astype(bf16)
    W1d = W_e1[D:2 * D].astype(bf16)
    W1e = W_e1[2 * D:].astype(bf16)
    h = _edge_mlp_call(xs, xd, ea_p,
                       W1s, W1d, W1e, b_e1.reshape(1, Hd),
                       W_e2.astype(bf16), b_e2.reshape(1, Hd),
                       g_e.reshape(1, Hd), be_ln.reshape(1, Hd))

    HC = Hd // _NC
    ZR = (-(-(N + 1) // _NS) + 7) // 8 * 8
    PADN = ZR * _NS
    zeros = jnp.zeros((ZR, HC), jnp.float32)
    aggp = _scatter_call(h, dsts2, zeros, N)
    agg = jnp.concatenate([aggp[:N], aggp[PADN:PADN + N]], axis=1)

    Wnx = W_n1[:D].astype(bf16)
    Wna = W_n1[D:].astype(bf16)
    return _node_mlp_call(x, agg, Wnx, Wna, b_n1.reshape(1, Hd),
                          W_n2.astype(bf16), b_n2.reshape(1, D),
                          g_n.reshape(1, D), bn_ln.reshape(1, D))


# speedup vs baseline: 1.8895x; 1.0007x over previous
import functools

import jax
import jax.numpy as jnp
from jax import lax
from jax.experimental import pallas as pl
from jax.experimental.pallas import tpu as pltpu
from jax.experimental.pallas import tpu_sc as plsc

_NC = 2
_NS = 16
_NW = _NC * _NS

_EP = 327680
_IW = 40
_IWG = 64
_CH = 8


def _silu(z):
    return z * jax.nn.sigmoid(z)


def _layer_norm(z, g, b):
    m = jnp.mean(z, axis=-1, keepdims=True)
    v = jnp.mean((z - m) ** 2, axis=-1, keepdims=True)
    return (z - m) * lax.rsqrt(v + 1e-5) * g + b


def _gather_call(x, src2, dst2):
    N, D = x.shape
    R, IW = src2.shape
    E = R * IW
    RW = R // _NW
    NIT = RW // _CH
    CE = _CH * IW

    mesh = plsc.VectorSubcoreMesh(core_axis_name="c", subcore_axis_name="s")

    @functools.partial(
        pl.kernel,
        out_type=[jax.ShapeDtypeStruct((E, D), jnp.float32),
                  jax.ShapeDtypeStruct((E, D), jnp.float32)],
        mesh=mesh,
        scratch_types=[
            pltpu.VMEM((_CH, IW), jnp.int32),
            pltpu.VMEM((_CH, IW), jnp.int32),
            pltpu.VMEM((CE, D), jnp.float32),
            pltpu.VMEM((CE, D), jnp.float32),
            pltpu.SemaphoreType.DMA,
        ],
    )
    def gather_kernel(x_hbm, src_hbm, dst_hbm, xs_hbm, xd_hbm,
                      idx_s, idx_d, rows_s, rows_d, sem):
        wid = lax.axis_index("s") * _NC + lax.axis_index("c")

        @pl.loop(0, NIT)
        def _chunk(i):
            row0 = wid * RW + i * _CH
            pltpu.sync_copy(src_hbm.at[pl.ds(row0, _CH)], idx_s)
            pltpu.sync_copy(dst_hbm.at[pl.ds(row0, _CH)], idx_d)
            copies = []
            for j in range(_CH):
                copies.append(pltpu.async_copy(
                    x_hbm.at[idx_s.at[j]], rows_s.at[pl.ds(j * IW, IW)], sem))
                copies.append(pltpu.async_copy(
                    x_hbm.at[idx_d.at[j]], rows_d.at[pl.ds(j * IW, IW)], sem))
            for c in copies:
                c.wait()
            e0 = row0 * IW
            pltpu.sync_copy(rows_s, xs_hbm.at[pl.ds(e0, CE)])
            pltpu.sync_copy(rows_d, xd_hbm.at[pl.ds(e0, CE)])

    return gather_kernel(x, src2, dst2)


def _scatter_call(h, dst2, zeros, N):
    E, Hd = h.shape
    R, IW = dst2.shape
    HC = Hd // _NC
    ZR = zeros.shape[0]
    PADN = ZR * _NS
    RW = R // _NS
    NIT = RW // _CH
    CE = _CH * IW

    mesh = plsc.VectorSubcoreMesh(core_axis_name="c", subcore_axis_name="s")

    @functools.partial(
        pl.kernel,
        out_type=jax.ShapeDtypeStruct((_NC * PADN, HC), jnp.float32),
        mesh=mesh,
        scratch_types=[
            pltpu.VMEM_SHARED((PADN, HC), jnp.float32),
            pltpu.VMEM((_CH, IW), jnp.int32),
            pltpu.VMEM((CE, HC), jnp.float32),
        ],
    )
    def scatter_kernel(h_hbm, dst_hbm, zero_hbm, agg_hbm,
                       acc, idx_v, rows_v):
        c = lax.axis_index("c")
        s = lax.axis_index("s")

        pltpu.sync_copy(zero_hbm, acc.at[pl.ds(s * ZR, ZR)])
        plsc.subcore_barrier()

        @pl.loop(0, NIT)
        def _chunk(i):
            row0 = s * RW + i * _CH
            pltpu.sync_copy(dst_hbm.at[pl.ds(row0, _CH)], idx_v)
            pltpu.sync_copy(h_hbm.at[pl.ds(row0 * IW, CE), pl.ds(c * HC, HC)],
                            rows_v)
            for j in range(_CH):
                pltpu.sync_copy(rows_v.at[pl.ds(j * IW, IW)],
                                acc.at[idx_v.at[j]], add=True)

        plsc.subcore_barrier()
        pltpu.sync_copy(acc.at[pl.ds(s * ZR, ZR)],
                        agg_hbm.at[pl.ds(c * PADN + s * ZR, ZR)])

    return scatter_kernel(h, dst2, zeros)


def _edge_mlp_call(xs, xd, ea, W1s, W1d, W1e, b1, W2, b2, g, be):
    E, D = xs.shape
    Hd = W2.shape[1]
    BE = 2048
    grid = (E // BE,)

    def body(xs_ref, xd_ref, ea_ref, W1s_ref, W1d_ref, W1e_ref, b1_ref,
             W2_ref, b2_ref, g_ref, be_ref, h_ref):
        f32 = jnp.float32
        z = (jnp.dot(xs_ref[...].astype(jnp.bfloat16), W1s_ref[...],
                     preferred_element_type=f32)
             + jnp.dot(xd_ref[...].astype(jnp.bfloat16), W1d_ref[...],
                       preferred_element_type=f32)
             + jnp.dot(ea_ref[...].astype(jnp.bfloat16), W1e_ref[...],
                       preferred_element_type=f32)
             + b1_ref[...])
        h1 = _silu(z).astype(jnp.bfloat16)
        h2 = jnp.dot(h1, W2_ref[...], preferred_element_type=f32) + b2_ref[...]
        h_ref[...] = _layer_norm(h2, g_ref[...], be_ref[...])

    full = lambda a: pl.BlockSpec(a.shape, lambda i: (0,) * a.ndim)
    return pl.pallas_call(
        body,
        grid=grid,
        in_specs=[
            pl.BlockSpec((BE, D), lambda i: (i, 0)),
            pl.BlockSpec((BE, D), lambda i: (i, 0)),
            pl.BlockSpec((BE, ea.shape[1]), lambda i: (i, 0)),
            full(W1s), full(W1d), full(W1e), full(b1),
            full(W2), full(b2), full(g), full(be),
        ],
        out_specs=pl.BlockSpec((BE, Hd), lambda i: (i, 0)),
        out_shape=jax.ShapeDtypeStruct((E, Hd), jnp.float32),
    )(xs, xd, ea, W1s, W1d, W1e, b1, W2, b2, g, be)


def _node_mlp_call(x, agg, Wnx, Wna, bn1, Wn2, bn2, g, b):
    N, D = x.shape
    Hd = agg.shape[1]
    BN = 1000
    grid = (N // BN,)

    def body(x_ref, agg_ref, Wnx_ref, Wna_ref, bn1_ref, Wn2_ref, bn2_ref,
             g_ref, b_ref, o_ref):
        f32 = jnp.float32
        z = (jnp.dot(x_ref[...].astype(jnp.bfloat16), Wnx_ref[...],
                     preferred_element_type=f32)
             + jnp.dot(agg_ref[...].astype(jnp.bfloat16), Wna_ref[...],
                       preferred_element_type=f32)
             + bn1_ref[...])
        u1 = _silu(z).astype(jnp.bfloat16)
        u = jnp.dot(u1, Wn2_ref[...], preferred_element_type=f32) + bn2_ref[...]
        o_ref[...] = x_ref[...] + _layer_norm(u, g_ref[...], b_ref[...])

    full = lambda a: pl.BlockSpec(a.shape, lambda i: (0,) * a.ndim)
    return pl.pallas_call(
        body,
        grid=grid,
        in_specs=[
            pl.BlockSpec((BN, D), lambda i: (i, 0)),
            pl.BlockSpec((BN, Hd), lambda i: (i, 0)),
            full(Wnx), full(Wna), full(bn1), full(Wn2), full(bn2),
            full(g), full(b),
        ],
        out_specs=pl.BlockSpec((BN, D), lambda i: (i, 0)),
        out_shape=jax.ShapeDtypeStruct((N, D), jnp.float32),
    )(x, agg, Wnx, Wna, bn1, Wn2, bn2, g, b)


def kernel(x, edge_index, edge_attr, W_e1, b_e1, W_e2, b_e2, g_e, be_ln,
           W_n1, b_n1, W_n2, b_n2, g_n, bn_ln):
    N, D = x.shape
    E = edge_index.shape[1]
    DE = edge_attr.shape[1]
    Hd = W_e2.shape[0]
    bf16 = jnp.bfloat16

    PAD = _EP - E
    src = edge_index[0]
    dst = edge_index[1]
    pad0 = jnp.zeros((PAD,), jnp.int32)
    src2 = jnp.concatenate([src, pad0]).reshape(_EP // _IW, _IW)
    dstg2 = jnp.concatenate([dst, pad0]).reshape(_EP // _IW, _IW)
    dsts2 = jnp.concatenate([dst, jnp.full((PAD,), N, jnp.int32)]).reshape(_EP // _IW, _IW)
    ea_p = jnp.concatenate([edge_attr, jnp.zeros((PAD, DE), edge_attr.dtype)])

    xs, xd = _gather_call(x, src2, dstg2)

    W1s = W_e1[:D].astype(bf16)
    W1d = W_e1[D:2 * D].astype(bf16)
    W1e = W_e1[2 * D:].astype(bf16)
    h = _edge_mlp_call(xs, xd, ea_p,
                       W1s, W1d, W1e, b_e1.reshape(1, Hd),
                       W_e2.astype(bf16), b_e2.reshape(1, Hd),
                       g_e.reshape(1, Hd), be_ln.reshape(1, Hd))

    HC = Hd // _NC
    ZR = (-(-(N + 1) // _NS) + 7) // 8 * 8
    PADN = ZR * _NS
    zeros = jnp.zeros((ZR, HC), jnp.float32)
    aggp = _scatter_call(h, dsts2, zeros, N)
    agg = jnp.concatenate([aggp[:N], aggp[PADN:PADN + N]], axis=1)

    Wnx = W_n1[:D].astype(bf16)
    Wna = W_n1[D:].astype(bf16)
    return _node_mlp_call(x, agg, Wnx, Wna, b_n1.reshape(1, Hd),
                          W_n2.astype(bf16), b_n2.reshape(1, D),
                          g_n.reshape(1, D), bn_ln.reshape(1, D))


# speedup vs baseline: 1.9772x; 1.0464x over previous
import functools

import jax
import jax.numpy as jnp
from jax import lax
from jax.experimental import pallas as pl
from jax.experimental.pallas import tpu as pltpu
from jax.experimental.pallas import tpu_sc as plsc

_NC = 2
_NS = 16
_NW = _NC * _NS

_EP = 327680
_IW = 40
_IWG = 64
_CH = 8


def _silu(z):
    return z * jax.nn.sigmoid(z)


def _layer_norm(z, g, b):
    m = jnp.mean(z, axis=-1, keepdims=True)
    v = jnp.mean((z - m) ** 2, axis=-1, keepdims=True)
    return (z - m) * lax.rsqrt(v + 1e-5) * g + b


def _gather_call(x, src2, dst2):
    N, D = x.shape
    R, IW = src2.shape
    E = R * IW
    RW = R // _NW
    NIT = RW // _CH
    CE = _CH * IW

    mesh = plsc.VectorSubcoreMesh(core_axis_name="c", subcore_axis_name="s")

    @functools.partial(
        pl.kernel,
        out_type=[jax.ShapeDtypeStruct((E, D), jnp.float32),
                  jax.ShapeDtypeStruct((E, D), jnp.float32)],
        mesh=mesh,
        scratch_types=[
            pltpu.VMEM((2, _CH, IW), jnp.int32),
            pltpu.VMEM((2, _CH, IW), jnp.int32),
            pltpu.VMEM((CE, D), jnp.float32),
            pltpu.VMEM((CE, D), jnp.float32),
            pltpu.SemaphoreType.DMA,
            pltpu.SemaphoreType.DMA,
            pltpu.SemaphoreType.DMA,
            pltpu.SemaphoreType.DMA,
        ],
    )
    def gather_kernel(x_hbm, src_hbm, dst_hbm, xs_hbm, xd_hbm,
                      idx_s, idx_d, rows_s, rows_d,
                      sem_i, sem_g, sem_ws, sem_wd):
        wid = lax.axis_index("s") * _NC + lax.axis_index("c")
        base = wid * RW

        def idx_load(i, b):
            row0 = base + i * _CH
            pltpu.make_async_copy(
                src_hbm.at[pl.ds(row0, _CH)], idx_s.at[b], sem_i).start()
            pltpu.make_async_copy(
                dst_hbm.at[pl.ds(row0, _CH)], idx_d.at[b], sem_i).start()

        def idx_wait(b):
            pltpu.make_async_copy(
                src_hbm.at[pl.ds(base, _CH)], idx_s.at[b], sem_i).wait()
            pltpu.make_async_copy(
                dst_hbm.at[pl.ds(base, _CH)], idx_d.at[b], sem_i).wait()

        def wb_wait(rows, out_hbm, sem):
            pltpu.make_async_copy(rows, out_hbm.at[pl.ds(0, CE)], sem).wait()

        idx_load(0, 0)

        @pl.loop(0, NIT, step=2)
        def _chunk(i):
            for b in range(2):
                ib = i + b

                idx_wait(b)

                @pl.when(ib + 1 < NIT)
                def _():
                    idx_load(ib + 1, 1 - b)

                e0 = (base + ib * _CH) * IW

                @pl.when(ib >= 1)
                def _():
                    wb_wait(rows_s, xs_hbm, sem_ws)
                gs = [pltpu.async_copy(x_hbm.at[idx_s.at[b, j]],
                                       rows_s.at[pl.ds(j * IW, IW)], sem_g)
                      for j in range(_CH)]
                for g in gs:
                    g.wait()
                pltpu.make_async_copy(
                    rows_s, xs_hbm.at[pl.ds(e0, CE)], sem_ws).start()

                @pl.when(ib >= 1)
                def _():
                    wb_wait(rows_d, xd_hbm, sem_wd)
                gd = [pltpu.async_copy(x_hbm.at[idx_d.at[b, j]],
                                       rows_d.at[pl.ds(j * IW, IW)], sem_g)
                      for j in range(_CH)]
                for g in gd:
                    g.wait()
                pltpu.make_async_copy(
                    rows_d, xd_hbm.at[pl.ds(e0, CE)], sem_wd).start()

        wb_wait(rows_s, xs_hbm, sem_ws)
        wb_wait(rows_d, xd_hbm, sem_wd)

    return gather_kernel(x, src2, dst2)


def _scatter_call(h, dst2, zeros, N):
    E, Hd = h.shape
    R, IW = dst2.shape
    HC = Hd // _NC
    ZR = zeros.shape[0]
    PADN = ZR * _NS
    RW = R // _NS
    NIT = RW // _CH
    CE = _CH * IW

    mesh = plsc.VectorSubcoreMesh(core_axis_name="c", subcore_axis_name="s")

    @functools.partial(
        pl.kernel,
        out_type=jax.ShapeDtypeStruct((_NC * PADN, HC), jnp.float32),
        mesh=mesh,
        scratch_types=[
            pltpu.VMEM_SHARED((PADN, HC), jnp.float32),
            pltpu.VMEM((_CH, IW), jnp.int32),
            pltpu.VMEM((CE, HC), jnp.float32),
        ],
    )
    def scatter_kernel(h_hbm, dst_hbm, zero_hbm, agg_hbm,
                       acc, idx_v, rows_v):
        c = lax.axis_index("c")
        s = lax.axis_index("s")

        pltpu.sync_copy(zero_hbm, acc.at[pl.ds(s * ZR, ZR)])
        plsc.subcore_barrier()

        @pl.loop(0, NIT)
        def _chunk(i):
            row0 = s * RW + i * _CH
            pltpu.sync_copy(dst_hbm.at[pl.ds(row0, _CH)], idx_v)
            pltpu.sync_copy(h_hbm.at[pl.ds(row0 * IW, CE), pl.ds(c * HC, HC)],
                            rows_v)
            for j in range(_CH):
                pltpu.sync_copy(rows_v.at[pl.ds(j * IW, IW)],
                                acc.at[idx_v.at[j]], add=True)

        plsc.subcore_barrier()
        pltpu.sync_copy(acc.at[pl.ds(s * ZR, ZR)],
                        agg_hbm.at[pl.ds(c * PADN + s * ZR, ZR)])

    return scatter_kernel(h, dst2, zeros)


def _edge_mlp_call(xs, xd, ea, W1s, W1d, W1e, b1, W2, b2, g, be):
    E, D = xs.shape
    Hd = W2.shape[1]
    BE = 2048
    grid = (E // BE,)

    def body(xs_ref, xd_ref, ea_ref, W1s_ref, W1d_ref, W1e_ref, b1_ref,
             W2_ref, b2_ref, g_ref, be_ref, h_ref):
        f32 = jnp.float32
        z = (jnp.dot(xs_ref[...].astype(jnp.bfloat16), W1s_ref[...],
                     preferred_element_type=f32)
             + jnp.dot(xd_ref[...].astype(jnp.bfloat16), W1d_ref[...],
                       preferred_element_type=f32)
             + jnp.dot(ea_ref[...].astype(jnp.bfloat16), W1e_ref[...],
                       preferred_element_type=f32)
             + b1_ref[...])
        h1 = _silu(z).astype(jnp.bfloat16)
        h2 = jnp.dot(h1, W2_ref[...], preferred_element_type=f32) + b2_ref[...]
        h_ref[...] = _layer_norm(h2, g_ref[...], be_ref[...])

    full = lambda a: pl.BlockSpec(a.shape, lambda i: (0,) * a.ndim)
    return pl.pallas_call(
        body,
        grid=grid,
        in_specs=[
            pl.BlockSpec((BE, D), lambda i: (i, 0)),
            pl.BlockSpec((BE, D), lambda i: (i, 0)),
            pl.BlockSpec((BE, ea.shape[1]), lambda i: (i, 0)),
            full(W1s), full(W1d), full(W1e), full(b1),
            full(W2), full(b2), full(g), full(be),
        ],
        out_specs=pl.BlockSpec((BE, Hd), lambda i: (i, 0)),
        out_shape=jax.ShapeDtypeStruct((E, Hd), jnp.float32),
    )(xs, xd, ea, W1s, W1d, W1e, b1, W2, b2, g, be)


def _node_mlp_call(x, agg, Wnx, Wna, bn1, Wn2, bn2, g, b):
    N, D = x.shape
    Hd = agg.shape[1]
    BN = 1000
    grid = (N // BN,)

    def body(x_ref, agg_ref, Wnx_ref, Wna_ref, bn1_ref, Wn2_ref, bn2_ref,
             g_ref, b_ref, o_ref):
        f32 = jnp.float32
        z = (jnp.dot(x_ref[...].astype(jnp.bfloat16), Wnx_ref[...],
                     preferred_element_type=f32)
             + jnp.dot(agg_ref[...].astype(jnp.bfloat16), Wna_ref[...],
                       preferred_element_type=f32)
             + bn1_ref[...])
        u1 = _silu(z).astype(jnp.bfloat16)
        u = jnp.dot(u1, Wn2_ref[...], preferred_element_type=f32) + bn2_ref[...]
        o_ref[...] = x_ref[...] + _layer_norm(u, g_ref[...], b_ref[...])

    full = lambda a: pl.BlockSpec(a.shape, lambda i: (0,) * a.ndim)
    return pl.pallas_call(
        body,
        grid=grid,
        in_specs=[
            pl.BlockSpec((BN, D), lambda i: (i, 0)),
            pl.BlockSpec((BN, Hd), lambda i: (i, 0)),
            full(Wnx), full(Wna), full(bn1), full(Wn2), full(bn2),
            full(g), full(b),
        ],
        out_specs=pl.BlockSpec((BN, D), lambda i: (i, 0)),
        out_shape=jax.ShapeDtypeStruct((N, D), jnp.float32),
    )(x, agg, Wnx, Wna, bn1, Wn2, bn2, g, b)


def kernel(x, edge_index, edge_attr, W_e1, b_e1, W_e2, b_e2, g_e, be_ln,
           W_n1, b_n1, W_n2, b_n2, g_n, bn_ln):
    N, D = x.shape
    E = edge_index.shape[1]
    DE = edge_attr.shape[1]
    Hd = W_e2.shape[0]
    bf16 = jnp.bfloat16

    PAD = _EP - E
    src = edge_index[0]
    dst = edge_index[1]
    pad0 = jnp.zeros((PAD,), jnp.int32)
    src2 = jnp.concatenate([src, pad0]).reshape(_EP // _IW, _IW)
    dstg2 = jnp.concatenate([dst, pad0]).reshape(_EP // _IW, _IW)
    dsts2 = jnp.concatenate([dst, jnp.full((PAD,), N, jnp.int32)]).reshape(_EP // _IW, _IW)
    ea_p = jnp.concatenate([edge_attr, jnp.zeros((PAD, DE), edge_attr.dtype)])

    xs, xd = _gather_call(x, src2, dstg2)

    W1s = W_e1[:D].astype(bf16)
    W1d = W_e1[D:2 * D].astype(bf16)
    W1e = W_e1[2 * D:].astype(bf16)
    h = _edge_mlp_call(xs, xd, ea_p,
                       W1s, W1d, W1e, b_e1.reshape(1, Hd),
                       W_e2.astype(bf16), b_e2.reshape(1, Hd),
                       g_e.reshape(1, Hd), be_ln.reshape(1, Hd))

    HC = Hd // _NC
    ZR = (-(-(N + 1) // _NS) + 7) // 8 * 8
    PADN = ZR * _NS
    zeros = jnp.zeros((ZR, HC), jnp.float32)
    aggp = _scatter_call(h, dsts2, zeros, N)
    agg = jnp.concatenate([aggp[:N], aggp[PADN:PADN + N]], axis=1)

    Wnx = W_n1[:D].astype(bf16)
    Wna = W_n1[D:].astype(bf16)
    return _node_mlp_call(x, agg, Wnx, Wna, b_n1.reshape(1, Hd),
                          W_n2.astype(bf16), b_n2.reshape(1, D),
                          g_n.reshape(1, D), bn_ln.reshape(1, D))
